# baseline (device time: 421499 ns/iter reference)
import jax
import jax.numpy as jnp
from jax import lax
from jax.experimental import pallas as pl
from jax.experimental.pallas import tpu as pltpu

N_DEV = 16


def kernel(x, w_mat, scale_x, scale_w):
    m_total, k_per = x.shape
    _, n = w_mat.shape
    blk_m = m_total // N_DEV

    def body(x_ref, w_ref, sx_ref, sw_ref, out_ref,
             comm_ref, send_sems, recv_sems, credit_sem):
        p = lax.axis_index("i")
        left = lax.rem(p + N_DEV - 1, N_DEV)
        right = lax.rem(p + 1, N_DEV)

        barrier = pltpu.get_barrier_semaphore()
        for nbr in (left, right):
            pl.semaphore_signal(barrier, inc=1, device_id=(nbr,),
                                device_id_type=pl.DeviceIdType.MESH)
        pl.semaphore_wait(barrier, 2)

        def partial_block(b):
            xb = x_ref[pl.ds(b * blk_m, blk_m), :]
            return lax.dot_general(
                xb, w_ref[:, :], (((1,), (0,)), ((), ())),
                preferred_element_type=jnp.int32)

        for s in range(N_DEV - 1):
            slot = s % 2
            b = lax.rem(p + 2 * N_DEV - 1 - s, N_DEV)
            part = partial_block(b)
            if s == 0:
                comm_ref[slot, :, :] = part
            else:
                comm_ref[slot, :, :] = comm_ref[slot, :, :] + part
            if s >= 1:
                pl.semaphore_wait(credit_sem, 1)
            rdma = pltpu.make_async_remote_copy(
                src_ref=comm_ref.at[slot],
                dst_ref=comm_ref.at[(s + 1) % 2],
                send_sem=send_sems.at[slot],
                recv_sem=recv_sems.at[(s + 1) % 2],
                device_id=(right,),
                device_id_type=pl.DeviceIdType.MESH,
            )
            rdma.start()
            rdma.wait()
            if s <= N_DEV - 3:
                pl.semaphore_signal(credit_sem, inc=1, device_id=(left,),
                                    device_id_type=pl.DeviceIdType.MESH)

        acc = comm_ref[(N_DEV - 1) % 2, :, :] + partial_block(p)
        scale = sx_ref[0] * sw_ref[0]
        out_ref[:, :] = jnp.maximum(acc.astype(jnp.float32) * scale, 0.0)

    return pl.pallas_call(
        body,
        out_shape=jax.ShapeDtypeStruct((blk_m, n), jnp.float32),
        in_specs=[
            pl.BlockSpec(memory_space=pltpu.VMEM),
            pl.BlockSpec(memory_space=pltpu.VMEM),
            pl.BlockSpec(memory_space=pltpu.SMEM),
            pl.BlockSpec(memory_space=pltpu.SMEM),
        ],
        out_specs=pl.BlockSpec(memory_space=pltpu.VMEM),
        scratch_shapes=[
            pltpu.VMEM((2, blk_m, n), jnp.int32),
            pltpu.SemaphoreType.DMA((2,)),
            pltpu.SemaphoreType.DMA((2,)),
            pltpu.SemaphoreType.REGULAR,
        ],
        compiler_params=pltpu.CompilerParams(collective_id=0),
    )(x, w_mat, scale_x, scale_w)


# device time: 206544 ns/iter; 2.0407x vs baseline; 2.0407x over previous
import jax
import jax.numpy as jnp
from jax import lax
from jax.experimental import pallas as pl
from jax.experimental.pallas import tpu as pltpu

N_DEV = 16
S = 4
C = 1


def kernel(x, w_mat, scale_x, scale_w):
    m_total, k_per = x.shape
    _, n = w_mat.shape
    blk_m = m_total // N_DEV
    n_half = n // 2
    sub_n = n_half // C
    n_steps = N_DEV - 1

    insts = []
    for c in range(C):
        insts.append({"dir": +1, "c0": c * sub_n})
        insts.append({"dir": -1, "c0": n_half + c * sub_n})

    def body(x_ref, w_ref, sx_ref, sw_ref, out_ref, *scratch):
        ni = len(insts)
        comm = scratch[:ni]
        send_sems = scratch[ni:2 * ni]
        recv_sems = scratch[2 * ni:3 * ni]
        credit = scratch[3 * ni:4 * ni]

        p = lax.axis_index("i")
        left = lax.rem(p + N_DEV - 1, N_DEV)
        right = lax.rem(p + 1, N_DEV)

        def peer_send(i):
            return right if insts[i]["dir"] == 1 else left

        def peer_recv(i):
            return left if insts[i]["dir"] == 1 else right

        barrier = pltpu.get_barrier_semaphore()
        for nbr in (left, right):
            pl.semaphore_signal(barrier, inc=1, device_id=(nbr,),
                                device_id_type=pl.DeviceIdType.MESH)
        pl.semaphore_wait(barrier, 2)

        def part_half(b, d):
            xb = x_ref[pl.ds(b * blk_m, blk_m), :]
            wh = w_ref[:, pl.ds(d * n_half, n_half)]
            return lax.dot_general(xb, wh, (((1,), (0,)), ((), ())),
                                   preferred_element_type=jnp.int32)

        def make_rdma(i, s):
            return pltpu.make_async_remote_copy(
                src_ref=comm[i].at[s % S],
                dst_ref=comm[i].at[(s + 1) % S],
                send_sem=send_sems[i].at[s % S],
                recv_sem=recv_sems[i].at[(s + 1) % S],
                device_id=(peer_send(i),),
                device_id_type=pl.DeviceIdType.MESH,
            )

        pending = [[None] * n_steps for _ in insts]

        for s in range(n_steps):
            sl = s % S
            parts = {}
            for d, inst0 in ((0, 0), (1, 1)):
                if insts[inst0]["dir"] == 1:
                    b = lax.rem(p + 2 * N_DEV - 1 - s, N_DEV)
                else:
                    b = lax.rem(p + 1 + s, N_DEV)
                parts[d] = part_half(b, d)
            for i in range(ni):
                d = 0 if insts[i]["c0"] < n_half else 1
                col0 = insts[i]["c0"] - d * n_half
                chunk = parts[d][:, col0:col0 + sub_n]
                if s == 0:
                    comm[i][sl, :, :] = chunk
                else:
                    make_rdma(i, s - 1).wait_recv()
                    pending[i][s - 1].wait_send()
                    comm[i][sl, :, :] = comm[i][sl, :, :] + chunk
                if s >= S - 1:
                    pl.semaphore_wait(credit[i], 1)
                rdma = make_rdma(i, s)
                rdma.start()
                pending[i][s] = rdma
                if 1 <= s <= N_DEV - S:
                    pl.semaphore_signal(credit[i], inc=1,
                                        device_id=(peer_recv(i),),
                                        device_id_type=pl.DeviceIdType.MESH)

        scale = sx_ref[0] * sw_ref[0]
        for d in (0, 1):
            partd = part_half(p, d)
            for i in range(ni):
                di = 0 if insts[i]["c0"] < n_half else 1
                if di != d:
                    continue
                make_rdma(i, n_steps - 1).wait_recv()
                pending[i][n_steps - 1].wait_send()
                col0 = insts[i]["c0"] - d * n_half
                acc = comm[i][n_steps % S, :, :] + partd[:, col0:col0 + sub_n]
                out_ref[:, pl.ds(insts[i]["c0"], sub_n)] = jnp.maximum(
                    acc.astype(jnp.float32) * scale, 0.0)

    scratch_shapes = (
        [pltpu.VMEM((S, blk_m, sub_n), jnp.int32) for _ in insts]
        + [pltpu.SemaphoreType.DMA((S,)) for _ in insts]
        + [pltpu.SemaphoreType.DMA((S,)) for _ in insts]
        + [pltpu.SemaphoreType.REGULAR for _ in insts]
    )

    return pl.pallas_call(
        body,
        out_shape=jax.ShapeDtypeStruct((blk_m, n), jnp.float32),
        in_specs=[
            pl.BlockSpec(memory_space=pltpu.VMEM),
            pl.BlockSpec(memory_space=pltpu.VMEM),
            pl.BlockSpec(memory_space=pltpu.SMEM),
            pl.BlockSpec(memory_space=pltpu.SMEM),
        ],
        out_specs=pl.BlockSpec(memory_space=pltpu.VMEM),
        scratch_shapes=scratch_shapes,
        compiler_params=pltpu.CompilerParams(collective_id=0),
    )(x, w_mat, scale_x, scale_w)


# device time: 181235 ns/iter; 2.3257x vs baseline; 1.1396x over previous
import jax
import jax.numpy as jnp
from jax import lax
from jax.experimental import pallas as pl
from jax.experimental.pallas import tpu as pltpu

N_DEV = 16
S = 4
C = 2


def kernel(x, w_mat, scale_x, scale_w):
    m_total, k_per = x.shape
    _, n = w_mat.shape
    blk_m = m_total // N_DEV
    n_half = n // 2
    sub_n = n_half // C
    n_steps = N_DEV - 1

    insts = []
    for c in range(C):
        insts.append({"dir": +1, "c0": c * sub_n})
        insts.append({"dir": -1, "c0": n_half + c * sub_n})

    def body(x_ref, w_ref, sx_ref, sw_ref, out_ref, *scratch):
        ni = len(insts)
        comm = scratch[:ni]
        send_sems = scratch[ni:2 * ni]
        recv_sems = scratch[2 * ni:3 * ni]
        credit = scratch[3 * ni:4 * ni]

        p = lax.axis_index("i")
        left = lax.rem(p + N_DEV - 1, N_DEV)
        right = lax.rem(p + 1, N_DEV)

        def peer_send(i):
            return right if insts[i]["dir"] == 1 else left

        def peer_recv(i):
            return left if insts[i]["dir"] == 1 else right

        barrier = pltpu.get_barrier_semaphore()
        for nbr in (left, right):
            pl.semaphore_signal(barrier, inc=1, device_id=(nbr,),
                                device_id_type=pl.DeviceIdType.MESH)
        pl.semaphore_wait(barrier, 2)

        def part_half(b, d):
            xb = x_ref[pl.ds(b * blk_m, blk_m), :]
            wh = w_ref[:, pl.ds(d * n_half, n_half)]
            return lax.dot_general(xb, wh, (((1,), (0,)), ((), ())),
                                   preferred_element_type=jnp.int32)

        def make_rdma(i, s):
            return pltpu.make_async_remote_copy(
                src_ref=comm[i].at[s % S],
                dst_ref=comm[i].at[(s + 1) % S],
                send_sem=send_sems[i].at[s % S],
                recv_sem=recv_sems[i].at[(s + 1) % S],
                device_id=(peer_send(i),),
                device_id_type=pl.DeviceIdType.MESH,
            )

        pending = [[None] * n_steps for _ in insts]

        for s in range(n_steps):
            sl = s % S
            parts = {}
            for d, inst0 in ((0, 0), (1, 1)):
                if insts[inst0]["dir"] == 1:
                    b = lax.rem(p + 2 * N_DEV - 1 - s, N_DEV)
                else:
                    b = lax.rem(p + 1 + s, N_DEV)
                parts[d] = part_half(b, d)
            for i in range(ni):
                d = 0 if insts[i]["c0"] < n_half else 1
                col0 = insts[i]["c0"] - d * n_half
                chunk = parts[d][:, col0:col0 + sub_n]
                if s == 0:
                    comm[i][sl, :, :] = chunk
                else:
                    make_rdma(i, s - 1).wait_recv()
                    pending[i][s - 1].wait_send()
                    comm[i][sl, :, :] = comm[i][sl, :, :] + chunk
                if s >= S - 1:
                    pl.semaphore_wait(credit[i], 1)
                rdma = make_rdma(i, s)
                rdma.start()
                pending[i][s] = rdma
                if 1 <= s <= N_DEV - S:
                    pl.semaphore_signal(credit[i], inc=1,
                                        device_id=(peer_recv(i),),
                                        device_id_type=pl.DeviceIdType.MESH)

        scale = sx_ref[0] * sw_ref[0]
        for d in (0, 1):
            partd = part_half(p, d)
            for i in range(ni):
                di = 0 if insts[i]["c0"] < n_half else 1
                if di != d:
                    continue
                make_rdma(i, n_steps - 1).wait_recv()
                pending[i][n_steps - 1].wait_send()
                col0 = insts[i]["c0"] - d * n_half
                acc = comm[i][n_steps % S, :, :] + partd[:, col0:col0 + sub_n]
                out_ref[:, pl.ds(insts[i]["c0"], sub_n)] = jnp.maximum(
                    acc.astype(jnp.float32) * scale, 0.0)

    scratch_shapes = (
        [pltpu.VMEM((S, blk_m, sub_n), jnp.int32) for _ in insts]
        + [pltpu.SemaphoreType.DMA((S,)) for _ in insts]
        + [pltpu.SemaphoreType.DMA((S,)) for _ in insts]
        + [pltpu.SemaphoreType.REGULAR for _ in insts]
    )

    return pl.pallas_call(
        body,
        out_shape=jax.ShapeDtypeStruct((blk_m, n), jnp.float32),
        in_specs=[
            pl.BlockSpec(memory_space=pltpu.VMEM),
            pl.BlockSpec(memory_space=pltpu.VMEM),
            pl.BlockSpec(memory_space=pltpu.SMEM),
            pl.BlockSpec(memory_space=pltpu.SMEM),
        ],
        out_specs=pl.BlockSpec(memory_space=pltpu.VMEM),
        scratch_shapes=scratch_shapes,
        compiler_params=pltpu.CompilerParams(collective_id=0),
    )(x, w_mat, scale_x, scale_w)
